# Initial kernel scaffold; baseline (speedup 1.0000x reference)
import jax
import jax.numpy as jnp
from jax.experimental import pallas as pl


def kernel(x, W_int, b_int, W_nh, b_nh, att_w_int, att_w_nh, W_dense, b_dense, src_int, dst_int, src_nh, dst_nh):
    raise NotImplementedError("write your pallas kernel here")


# speedup vs baseline: 25.9659x; 25.9659x over previous
import functools

import jax
import jax.numpy as jnp
from jax import lax
from jax.experimental import pallas as pl
from jax.experimental.pallas import tpu as pltpu
from jax.experimental.pallas import tpu_sc as plsc

L = 2048
B = 64
V = 11
F = 16
E = L * 16
BV = B * V
BVP = 768
BF = B * F
NC = 2
NS = 16
SLAB = 32
NPASS = (L // NS) // SLAB


def _sc_count_matrices(flat):
    mesh = plsc.VectorSubcoreMesh(core_axis_name="c", subcore_axis_name="s")

    @functools.partial(
        pl.kernel,
        out_type=jax.ShapeDtypeStruct((NC, L, L), jnp.float32),
        mesh=mesh,
        scratch_types=[
            pltpu.VMEM((E,), jnp.int32),
            pltpu.VMEM((SLAB, L), jnp.float32),
        ],
        compiler_params=pltpu.CompilerParams(needs_layout_passes=False),
    )
    def k(flat_hbm, out_hbm, fbuf, slab):
        c = lax.axis_index("c")
        s = lax.axis_index("s")

        pltpu.sync_copy(flat_hbm.at[c], fbuf)

        for p in range(NPASS):
            row0 = s * (SLAB * NPASS) + p * SLAB
            base = row0 * L

            def zero_body(i, carry):
                r = lax.shift_right_logical(i, 7)
                col = lax.mul(jnp.bitwise_and(i, 127), 16)
                slab[r, pl.ds(col, 16)] = jnp.zeros((16,), jnp.float32)
                return carry
            lax.fori_loop(0, SLAB * (L // 16), zero_body, 0)

            def edge_body(i, carry):
                f = fbuf[pl.ds(i * 16, 16)]
                loc = f - base
                inr = (loc >= 0) & (loc < SLAB * L)
                locc = jnp.where(inr, loc, 0)
                cnt, last = plsc.scan_count(locc, inr)
                r_idx = lax.shift_right_logical(locc, 11)
                c_idx = jnp.bitwise_and(locc, L - 1)
                plsc.addupdate_scatter(
                    slab, [r_idx, c_idx], cnt.astype(jnp.float32),
                    mask=last & inr)
                return carry
            lax.fori_loop(0, E // 16, edge_body, 0)

            pltpu.sync_copy(slab, out_hbm.at[c, pl.ds(row0, SLAB)])

    return k(flat)


def _epilogue_grid_body(m_ref, x_ref, wx, bt, wt, g, gt, wd, bd, out):
    gi = pl.program_id(0)
    gm = g[...]
    dn = (((1,), (0,)), ((), ()))

    agg = lax.dot_general(m_ref[0], x_ref[...], dn,
                          preferred_element_type=jnp.float32)
    z = lax.dot_general(agg, wx[0], dn,
                        preferred_element_type=jnp.float32)
    h = jnp.tanh(z + bt[0])
    t = jnp.tanh(lax.dot_general(h * wt[0], gm, dn,
                                 preferred_element_type=jnp.float32))
    ex = jnp.exp(t)
    sc = ex / jnp.sum(ex, axis=0, keepdims=True)
    sexp = lax.dot_general(sc, gt[...], dn,
                           preferred_element_type=jnp.float32)
    rep = jnp.sum(h * sexp, axis=0, keepdims=True)
    contrib = lax.dot_general(rep * wd[0], gm, dn,
                              preferred_element_type=jnp.float32)

    @pl.when(gi == 0)
    def _():
        out[...] = bd[...] + contrib

    @pl.when(gi == 1)
    def _():
        out[...] = out[...] + contrib


def kernel(x, W_int, b_int, W_nh, b_nh, att_w_int, att_w_nh, W_dense, b_dense,
           src_int, dst_int, src_nh, dst_nh):
    x2d = jnp.concatenate(
        [x.reshape(L, BV), jnp.zeros((L, BVP - BV), jnp.float32)], axis=1)
    flat = jnp.stack([
        dst_int.astype(jnp.int32) * L + src_int.astype(jnp.int32),
        dst_nh.astype(jnp.int32) * L + src_nh.astype(jnp.int32),
    ])

    m2 = _sc_count_matrices(flat)

    eyeB = jnp.eye(B, dtype=jnp.float32)
    pad_rows = jnp.zeros((BVP - BV, BF), jnp.float32)
    wx = jnp.stack([
        jnp.concatenate(
            [jnp.einsum('bc,vf->bvcf', eyeB, W_int).reshape(BV, BF), pad_rows]),
        jnp.concatenate(
            [jnp.einsum('bc,vf->bvcf', eyeB, W_nh).reshape(BV, BF), pad_rows]),
    ])
    bt = jnp.stack([jnp.tile(b_int, B), jnp.tile(b_nh, B)]).reshape(NC, 1, BF)
    wt = jnp.stack([jnp.tile(att_w_int, B),
                    jnp.tile(att_w_nh, B)]).reshape(NC, 1, BF)
    g = jnp.repeat(eyeB, F, axis=0)
    gt = g.T
    wd = jnp.stack([jnp.tile(W_dense[:F, 0], B),
                    jnp.tile(W_dense[F:, 0], B)]).reshape(NC, 1, BF)
    bd = b_dense.reshape(1, 1)

    out = pl.pallas_call(
        _epilogue_grid_body,
        grid=(NC,),
        in_specs=[
            pl.BlockSpec((1, L, L), lambda i: (i, 0, 0)),
            pl.BlockSpec((L, BVP), lambda i: (0, 0)),
            pl.BlockSpec((1, BVP, BF), lambda i: (i, 0, 0)),
            pl.BlockSpec((1, 1, BF), lambda i: (i, 0, 0)),
            pl.BlockSpec((1, 1, BF), lambda i: (i, 0, 0)),
            pl.BlockSpec((BF, B), lambda i: (0, 0)),
            pl.BlockSpec((B, BF), lambda i: (0, 0)),
            pl.BlockSpec((1, 1, BF), lambda i: (i, 0, 0)),
            pl.BlockSpec((1, 1), lambda i: (0, 0)),
        ],
        out_specs=pl.BlockSpec((1, B), lambda i: (0, 0)),
        out_shape=jax.ShapeDtypeStruct((1, B), jnp.float32),
    )(m2, x2d, wx, bt, wt, g, gt, wd, bd)
    return out.reshape(B)


# speedup vs baseline: 51.4556x; 1.9817x over previous
import functools

import jax
import jax.numpy as jnp
from jax import lax
from jax.experimental import pallas as pl
from jax.experimental.pallas import tpu as pltpu
from jax.experimental.pallas import tpu_sc as plsc

L = 2048
B = 64
V = 11
F = 16
E = L * 16
BV = B * V
BVP = 768
BF = B * F
NC = 2
NS = 16
SLAB = 32
NPASS = (L // NS) // SLAB


def _sc_count_matrices(flat):
    mesh = plsc.VectorSubcoreMesh(core_axis_name="c", subcore_axis_name="s")

    @functools.partial(
        pl.kernel,
        out_type=jax.ShapeDtypeStruct((NC, L, L), jnp.float32),
        mesh=mesh,
        scratch_types=[
            pltpu.VMEM((E,), jnp.int32),
            pltpu.VMEM((SLAB, L), jnp.float32),
        ],
        compiler_params=pltpu.CompilerParams(needs_layout_passes=False),
    )
    def k(flat_hbm, out_hbm, fbuf, slab):
        c = lax.axis_index("c")
        s = lax.axis_index("s")

        pltpu.sync_copy(flat_hbm.at[c], fbuf)

        for p in range(NPASS):
            row0 = s * (SLAB * NPASS) + p * SLAB
            base = row0 * L

            @plsc.parallel_loop(0, SLAB * (L // 16), unroll=8)
            def _zero(i):
                r = lax.shift_right_logical(i, 7)
                col = lax.mul(jnp.bitwise_and(i, 127), 16)
                slab[r, pl.ds(col, 16)] = jnp.zeros((16,), jnp.float32)

            @plsc.parallel_loop(0, E // 16, unroll=8)
            def _edges(i):
                f = fbuf[pl.ds(i * 16, 16)]
                loc = f - base
                inr = (loc >= 0) & (loc < SLAB * L)
                locc = jnp.where(inr, loc, 0)
                cnt, last = plsc.scan_count(locc, inr)
                r_idx = lax.shift_right_logical(locc, 11)
                c_idx = jnp.bitwise_and(locc, L - 1)
                plsc.addupdate_scatter(
                    slab, [r_idx, c_idx], cnt.astype(jnp.float32),
                    mask=last & inr)

            pltpu.sync_copy(slab, out_hbm.at[c, pl.ds(row0, SLAB)])

    return k(flat)


def _epilogue_grid_body(m_ref, x_ref, wx, bt, wt, g, gt, wd, bd, out):
    gi = pl.program_id(0)
    gm = g[...]
    dn = (((1,), (0,)), ((), ()))

    agg = lax.dot_general(m_ref[0], x_ref[...], dn,
                          preferred_element_type=jnp.float32)
    z = lax.dot_general(agg, wx[0], dn,
                        preferred_element_type=jnp.float32)
    h = jnp.tanh(z + bt[0])
    t = jnp.tanh(lax.dot_general(h * wt[0], gm, dn,
                                 preferred_element_type=jnp.float32))
    ex = jnp.exp(t)
    sc = ex / jnp.sum(ex, axis=0, keepdims=True)
    sexp = lax.dot_general(sc, gt[...], dn,
                           preferred_element_type=jnp.float32)
    rep = jnp.sum(h * sexp, axis=0, keepdims=True)
    contrib = lax.dot_general(rep * wd[0], gm, dn,
                              preferred_element_type=jnp.float32)

    @pl.when(gi == 0)
    def _():
        out[...] = bd[...] + contrib

    @pl.when(gi == 1)
    def _():
        out[...] = out[...] + contrib


def kernel(x, W_int, b_int, W_nh, b_nh, att_w_int, att_w_nh, W_dense, b_dense,
           src_int, dst_int, src_nh, dst_nh):
    x2d = jnp.concatenate(
        [x.reshape(L, BV), jnp.zeros((L, BVP - BV), jnp.float32)], axis=1)
    flat = jnp.stack([
        dst_int.astype(jnp.int32) * L + src_int.astype(jnp.int32),
        dst_nh.astype(jnp.int32) * L + src_nh.astype(jnp.int32),
    ])

    m2 = _sc_count_matrices(flat)

    eyeB = jnp.eye(B, dtype=jnp.float32)
    pad_rows = jnp.zeros((BVP - BV, BF), jnp.float32)
    wx = jnp.stack([
        jnp.concatenate(
            [jnp.einsum('bc,vf->bvcf', eyeB, W_int).reshape(BV, BF), pad_rows]),
        jnp.concatenate(
            [jnp.einsum('bc,vf->bvcf', eyeB, W_nh).reshape(BV, BF), pad_rows]),
    ])
    bt = jnp.stack([jnp.tile(b_int, B), jnp.tile(b_nh, B)]).reshape(NC, 1, BF)
    wt = jnp.stack([jnp.tile(att_w_int, B),
                    jnp.tile(att_w_nh, B)]).reshape(NC, 1, BF)
    g = jnp.repeat(eyeB, F, axis=0)
    gt = g.T
    wd = jnp.stack([jnp.tile(W_dense[:F, 0], B),
                    jnp.tile(W_dense[F:, 0], B)]).reshape(NC, 1, BF)
    bd = b_dense.reshape(1, 1)

    out = pl.pallas_call(
        _epilogue_grid_body,
        grid=(NC,),
        in_specs=[
            pl.BlockSpec((1, L, L), lambda i: (i, 0, 0)),
            pl.BlockSpec((L, BVP), lambda i: (0, 0)),
            pl.BlockSpec((1, BVP, BF), lambda i: (i, 0, 0)),
            pl.BlockSpec((1, 1, BF), lambda i: (i, 0, 0)),
            pl.BlockSpec((1, 1, BF), lambda i: (i, 0, 0)),
            pl.BlockSpec((BF, B), lambda i: (0, 0)),
            pl.BlockSpec((B, BF), lambda i: (0, 0)),
            pl.BlockSpec((1, 1, BF), lambda i: (i, 0, 0)),
            pl.BlockSpec((1, 1), lambda i: (0, 0)),
        ],
        out_specs=pl.BlockSpec((1, B), lambda i: (0, 0)),
        out_shape=jax.ShapeDtypeStruct((1, B), jnp.float32),
    )(m2, x2d, wx, bt, wt, g, gt, wd, bd)
    return out.reshape(B)


# speedup vs baseline: 52.8890x; 1.0279x over previous
import functools

import jax
import jax.numpy as jnp
from jax import lax
from jax.experimental import pallas as pl
from jax.experimental.pallas import tpu as pltpu
from jax.experimental.pallas import tpu_sc as plsc

L = 2048
B = 64
V = 11
F = 16
E = L * 16
BV = B * V
BVP = 768
BF = B * F
NC = 2
NS = 16
SLAB = 32
NPASS = (L // NS) // SLAB


def _sc_count_matrices(flat_i, flat_n):
    mesh = plsc.VectorSubcoreMesh(core_axis_name="c", subcore_axis_name="s")

    @functools.partial(
        pl.kernel,
        out_type=jax.ShapeDtypeStruct((NC, L, L), jnp.float32),
        mesh=mesh,
        scratch_types=[
            pltpu.VMEM((E,), jnp.int32),
            pltpu.VMEM((SLAB, L), jnp.float32),
        ],
        compiler_params=pltpu.CompilerParams(needs_layout_passes=False),
    )
    def k(fi_hbm, fn_hbm, out_hbm, fbuf, slab):
        c = lax.axis_index("c")
        s = lax.axis_index("s")

        @pl.when(c == 0)
        def _():
            pltpu.sync_copy(fi_hbm, fbuf)

        @pl.when(c == 1)
        def _():
            pltpu.sync_copy(fn_hbm, fbuf)

        for p in range(NPASS):
            row0 = s * (SLAB * NPASS) + p * SLAB
            base = row0 * L

            @plsc.parallel_loop(0, SLAB * (L // 16), unroll=8)
            def _zero(i):
                r = lax.shift_right_logical(i, 7)
                col = lax.mul(jnp.bitwise_and(i, 127), 16)
                slab[r, pl.ds(col, 16)] = jnp.zeros((16,), jnp.float32)

            @plsc.parallel_loop(0, E // 16, unroll=16)
            def _edges(i):
                f = fbuf[pl.ds(i * 16, 16)]
                loc = f - base
                inr = (loc >= 0) & (loc < SLAB * L)
                locc = jnp.where(inr, loc, 0)
                cnt, last = plsc.scan_count(locc, inr)
                r_idx = lax.shift_right_logical(locc, 11)
                c_idx = jnp.bitwise_and(locc, L - 1)
                plsc.addupdate_scatter(
                    slab, [r_idx, c_idx], cnt.astype(jnp.float32),
                    mask=last & inr)

            pltpu.sync_copy(slab, out_hbm.at[c, pl.ds(row0, SLAB)])

    return k(flat_i, flat_n)


def _epilogue_grid_body(m_ref, x_ref, wx, bt, wt, g, gt, wd, bd, out):
    gi = pl.program_id(0)
    gm = g[...]
    dn = (((1,), (0,)), ((), ()))

    agg = lax.dot_general(m_ref[0].astype(jnp.bfloat16),
                          x_ref[...].astype(jnp.bfloat16), dn,
                          preferred_element_type=jnp.float32)
    z = lax.dot_general(agg, wx[0], dn,
                        preferred_element_type=jnp.float32)
    h = jnp.tanh(z + bt[0])
    t = jnp.tanh(lax.dot_general(h * wt[0], gm, dn,
                                 preferred_element_type=jnp.float32))
    ex = jnp.exp(t)
    sc = ex / jnp.sum(ex, axis=0, keepdims=True)
    sexp = lax.dot_general(sc, gt[...], dn,
                           preferred_element_type=jnp.float32)
    rep = jnp.sum(h * sexp, axis=0, keepdims=True)
    contrib = lax.dot_general(rep * wd[0], gm, dn,
                              preferred_element_type=jnp.float32)

    @pl.when(gi == 0)
    def _():
        out[...] = bd[...] + contrib

    @pl.when(gi == 1)
    def _():
        out[...] = out[...] + contrib


def kernel(x, W_int, b_int, W_nh, b_nh, att_w_int, att_w_nh, W_dense, b_dense,
           src_int, dst_int, src_nh, dst_nh):
    x2d = x.reshape(L, BV)
    flat_i = dst_int.astype(jnp.int32) * L + src_int.astype(jnp.int32)
    flat_n = dst_nh.astype(jnp.int32) * L + src_nh.astype(jnp.int32)

    m2 = _sc_count_matrices(flat_i, flat_n)

    eyeB = jnp.eye(B, dtype=jnp.float32)
    wx = jnp.stack([
        jnp.einsum('bc,vf->bvcf', eyeB, W_int).reshape(BV, BF),
        jnp.einsum('bc,vf->bvcf', eyeB, W_nh).reshape(BV, BF),
    ])
    bt = jnp.stack([jnp.tile(b_int, B), jnp.tile(b_nh, B)]).reshape(NC, 1, BF)
    wt = jnp.stack([jnp.tile(att_w_int, B),
                    jnp.tile(att_w_nh, B)]).reshape(NC, 1, BF)
    g = jnp.repeat(eyeB, F, axis=0)
    gt = g.T
    wd = jnp.stack([jnp.tile(W_dense[:F, 0], B),
                    jnp.tile(W_dense[F:, 0], B)]).reshape(NC, 1, BF)
    bd = b_dense.reshape(1, 1)

    out = pl.pallas_call(
        _epilogue_grid_body,
        grid=(NC,),
        in_specs=[
            pl.BlockSpec((1, L, L), lambda i: (i, 0, 0)),
            pl.BlockSpec((L, BV), lambda i: (0, 0)),
            pl.BlockSpec((1, BV, BF), lambda i: (i, 0, 0)),
            pl.BlockSpec((1, 1, BF), lambda i: (i, 0, 0)),
            pl.BlockSpec((1, 1, BF), lambda i: (i, 0, 0)),
            pl.BlockSpec((BF, B), lambda i: (0, 0)),
            pl.BlockSpec((B, BF), lambda i: (0, 0)),
            pl.BlockSpec((1, 1, BF), lambda i: (i, 0, 0)),
            pl.BlockSpec((1, 1), lambda i: (0, 0)),
        ],
        out_specs=pl.BlockSpec((1, B), lambda i: (0, 0)),
        out_shape=jax.ShapeDtypeStruct((1, B), jnp.float32),
    )(m2, x2d, wx, bt, wt, g, gt, wd, bd)
    return out.reshape(B)


# speedup vs baseline: 53.9013x; 1.0191x over previous
import functools

import jax
import jax.numpy as jnp
from jax import lax
from jax.experimental import pallas as pl
from jax.experimental.pallas import tpu as pltpu
from jax.experimental.pallas import tpu_sc as plsc

L = 2048
B = 64
V = 11
F = 16
E = L * 16
BV = B * V
BVP = 768
BF = B * F
NC = 2
NS = 16
SLAB = 32
NPASS = (L // NS) // SLAB


def _sc_count_matrices(flat_i, flat_n):
    mesh = plsc.VectorSubcoreMesh(core_axis_name="c", subcore_axis_name="s")

    @functools.partial(
        pl.kernel,
        out_type=jax.ShapeDtypeStruct((NC, L, L), jnp.float32),
        mesh=mesh,
        scratch_types=[
            pltpu.VMEM((E,), jnp.int32),
            pltpu.VMEM((SLAB, L), jnp.float32),
        ],
        compiler_params=pltpu.CompilerParams(needs_layout_passes=False),
    )
    def k(fi_hbm, fn_hbm, out_hbm, fbuf, slab):
        c = lax.axis_index("c")
        s = lax.axis_index("s")

        @pl.when(c == 0)
        def _():
            pltpu.sync_copy(fi_hbm, fbuf)

        @pl.when(c == 1)
        def _():
            pltpu.sync_copy(fn_hbm, fbuf)

        for p in range(NPASS):
            row0 = s * (SLAB * NPASS) + p * SLAB
            base = row0 * L

            @plsc.parallel_loop(0, SLAB * (L // 16), unroll=8)
            def _zero(i):
                r = lax.shift_right_logical(i, 7)
                col = lax.mul(jnp.bitwise_and(i, 127), 16)
                slab[r, pl.ds(col, 16)] = jnp.zeros((16,), jnp.float32)

            @plsc.parallel_loop(0, E // 16, unroll=8)
            def _edges(i):
                f = fbuf[pl.ds(i * 16, 16)]
                loc = f - base
                inr = (loc >= 0) & (loc < SLAB * L)
                locc = jnp.where(inr, loc, 0)
                cnt, last = plsc.scan_count(locc, inr)
                r_idx = lax.shift_right_logical(locc, 11)
                c_idx = jnp.bitwise_and(locc, L - 1)
                plsc.addupdate_scatter(
                    slab, [r_idx, c_idx], cnt.astype(jnp.float32),
                    mask=last & inr)

            pltpu.sync_copy(slab, out_hbm.at[c, pl.ds(row0, SLAB)])

    return k(flat_i, flat_n)


def _epilogue_grid_body(m_ref, x_ref, wx, bt, wt, g, gt, wd, bd, out):
    gi = pl.program_id(0)
    gm = g[...]
    dn = (((1,), (0,)), ((), ()))

    agg = lax.dot_general(m_ref[0].astype(jnp.bfloat16),
                          x_ref[...].astype(jnp.bfloat16), dn,
                          preferred_element_type=jnp.float32)
    z = lax.dot_general(agg, wx[0], dn,
                        preferred_element_type=jnp.float32)
    h = jnp.tanh(z + bt[0])
    t = jnp.tanh(lax.dot_general(h * wt[0], gm, dn,
                                 preferred_element_type=jnp.float32))
    ex = jnp.exp(t)
    sc = ex / jnp.sum(ex, axis=0, keepdims=True)
    sexp = lax.dot_general(sc, gt[...], dn,
                           preferred_element_type=jnp.float32)
    rep = jnp.sum(h * sexp, axis=0, keepdims=True)
    contrib = lax.dot_general(rep * wd[0], gm, dn,
                              preferred_element_type=jnp.float32)

    @pl.when(gi == 0)
    def _():
        out[...] = bd[...] + contrib

    @pl.when(gi == 1)
    def _():
        out[...] = out[...] + contrib


def kernel(x, W_int, b_int, W_nh, b_nh, att_w_int, att_w_nh, W_dense, b_dense,
           src_int, dst_int, src_nh, dst_nh):
    x2d = x.reshape(L, BV)
    flat_i = dst_int.astype(jnp.int32) * L + src_int.astype(jnp.int32)
    flat_n = dst_nh.astype(jnp.int32) * L + src_nh.astype(jnp.int32)

    m2 = _sc_count_matrices(flat_i, flat_n)

    eyeB = jnp.eye(B, dtype=jnp.float32)
    wx = jnp.stack([
        jnp.einsum('bc,vf->bvcf', eyeB, W_int).reshape(BV, BF),
        jnp.einsum('bc,vf->bvcf', eyeB, W_nh).reshape(BV, BF),
    ])
    bt = jnp.stack([jnp.tile(b_int, B), jnp.tile(b_nh, B)]).reshape(NC, 1, BF)
    wt = jnp.stack([jnp.tile(att_w_int, B),
                    jnp.tile(att_w_nh, B)]).reshape(NC, 1, BF)
    g = jnp.repeat(eyeB, F, axis=0)
    gt = g.T
    wd = jnp.stack([jnp.tile(W_dense[:F, 0], B),
                    jnp.tile(W_dense[F:, 0], B)]).reshape(NC, 1, BF)
    bd = b_dense.reshape(1, 1)

    out = pl.pallas_call(
        _epilogue_grid_body,
        grid=(NC,),
        in_specs=[
            pl.BlockSpec((1, L, L), lambda i: (i, 0, 0)),
            pl.BlockSpec((L, BV), lambda i: (0, 0)),
            pl.BlockSpec((1, BV, BF), lambda i: (i, 0, 0)),
            pl.BlockSpec((1, 1, BF), lambda i: (i, 0, 0)),
            pl.BlockSpec((1, 1, BF), lambda i: (i, 0, 0)),
            pl.BlockSpec((BF, B), lambda i: (0, 0)),
            pl.BlockSpec((B, BF), lambda i: (0, 0)),
            pl.BlockSpec((1, 1, BF), lambda i: (i, 0, 0)),
            pl.BlockSpec((1, 1), lambda i: (0, 0)),
        ],
        out_specs=pl.BlockSpec((1, B), lambda i: (0, 0)),
        out_shape=jax.ShapeDtypeStruct((1, B), jnp.float32),
    )(m2, x2d, wx, bt, wt, g, gt, wd, bd)
    return out.reshape(B)
